# baseline (speedup 1.0000x reference)
import jax
import jax.numpy as jnp
from jax.experimental import pallas as pl
from jax.experimental.pallas import tpu as pltpu

_IN = 10
_OUT = 5
_TBT = 1024


def _linear_tiles_kernel(x_ref, wt_ref, b_ref, o_ref):
    t = x_ref.shape[0]
    x2 = x_ref[...].reshape(t * 8, _IN)
    y = jnp.dot(x2, wt_ref[...], preferred_element_type=jnp.float32)
    o_ref[...] = (y + b_ref[...]).astype(o_ref.dtype)


@jax.jit
def _forward(x, w, b):
    B = x.shape[0]
    Bp = ((B + 7) // 8) * 8
    if Bp != B:
        x = jnp.pad(x, ((0, Bp - B), (0, 0)))
    T = Bp // 8
    xv = x.reshape(T, 8, _IN)

    wt = w.T.astype(x.dtype)
    b2 = b.reshape(1, _OUT).astype(x.dtype)

    cost = pl.CostEstimate(
        flops=2 * Bp * _IN * _OUT,
        transcendentals=0,
        bytes_accessed=T * 2 * 8 * 128 * 4,
    )

    out = pl.pallas_call(
        _linear_tiles_kernel,
        out_shape=jax.ShapeDtypeStruct((Bp, _OUT), x.dtype),
        grid=(pl.cdiv(T, _TBT),),
        in_specs=[
            pl.BlockSpec((_TBT, 8, _IN), lambda i: (i, 0, 0)),
            pl.BlockSpec((_IN, _OUT), lambda i: (0, 0)),
            pl.BlockSpec((1, _OUT), lambda i: (0, 0)),
        ],
        out_specs=pl.BlockSpec((_TBT * 8, _OUT), lambda i: (i, 0)),
        cost_estimate=cost,
        compiler_params=pltpu.CompilerParams(
            dimension_semantics=("parallel",),
        ),
    )(xv, wt, b2)

    return out[:B]


def kernel(x, w, b):
    return _forward(x, w, b)


# speedup vs baseline: 1.2262x; 1.2262x over previous
import jax
import jax.numpy as jnp
from jax.experimental import pallas as pl
from jax.experimental.pallas import tpu as pltpu

_IN = 10
_OUT = 5
_TBT = 2048


def _linear_tiles_kernel(x_ref, wt_ref, b_ref, o_ref):
    t = x_ref.shape[0]
    x2 = x_ref[...].reshape(t * 8, _IN)
    y = jnp.dot(x2, wt_ref[...], preferred_element_type=jnp.float32)
    o_ref[...] = (y + b_ref[...]).reshape(t, 8, _OUT).astype(o_ref.dtype)


@jax.jit
def _forward(x, w, b):
    B = x.shape[0]
    Bp = ((B + 7) // 8) * 8
    if Bp != B:
        x = jnp.pad(x, ((0, Bp - B), (0, 0)))
    T = Bp // 8
    xv = x.reshape(T, 8, _IN)

    wt = w.T.astype(x.dtype)
    b2 = b.reshape(1, _OUT).astype(x.dtype)

    cost = pl.CostEstimate(
        flops=2 * Bp * _IN * _OUT,
        transcendentals=0,
        bytes_accessed=T * 2 * 8 * 128 * 4,
    )

    out = pl.pallas_call(
        _linear_tiles_kernel,
        out_shape=jax.ShapeDtypeStruct((T, 8, _OUT), x.dtype),
        grid=(pl.cdiv(T, _TBT),),
        in_specs=[
            pl.BlockSpec((_TBT, 8, _IN), lambda i: (i, 0, 0)),
            pl.BlockSpec((_IN, _OUT), lambda i: (0, 0)),
            pl.BlockSpec((1, _OUT), lambda i: (0, 0)),
        ],
        out_specs=pl.BlockSpec((_TBT, 8, _OUT), lambda i: (i, 0, 0)),
        cost_estimate=cost,
        compiler_params=pltpu.CompilerParams(
            dimension_semantics=("parallel",),
        ),
    )(xv, wt, b2)

    return out.reshape(Bp, _OUT)[:B]


def kernel(x, w, b):
    return _forward(x, w, b)
